# baseline (speedup 1.0000x reference)
import functools

import jax
import jax.numpy as jnp
from jax import lax
from jax.experimental import pallas as pl
from jax.experimental.pallas import tpu as pltpu
from jax.experimental.pallas import tpu_sc as plsc

NUM_EMB = 1000
EMB_DIM = 1024
BATCH = 16384

B_SC = 10240
B_TC = BATCH - B_SC

_info = plsc.get_sparse_core_info()
NC, NS = _info.num_cores, _info.num_subcores
NW = NC * NS
B_PER_W = B_SC // NW
CHUNK = 32
NCH = B_PER_W // CHUNK
NBUF = 3


def _gather_body(idx_hbm, table_hbm, out_hbm, idx_v, rows_v,
                 g0, g1, g2, w0, w1, w2):
    gsems = (g0, g1, g2)
    wsems = (w0, w1, w2)
    wid = lax.axis_index("s") * NC + lax.axis_index("c")
    base = wid * B_PER_W
    pltpu.sync_copy(idx_hbm.at[pl.ds(base, B_PER_W)], idx_v)

    gh = [None] * NBUF
    wh = [None] * NBUF
    for ch in range(NBUF):
        b = ch % NBUF
        gh[b] = pltpu.async_copy(
            table_hbm.at[idx_v.at[pl.ds(ch * CHUNK, CHUNK)]],
            rows_v.at[b], gsems[b])
    for ch in range(NCH):
        b = ch % NBUF
        gh[b].wait()
        wh[b] = pltpu.async_copy(rows_v.at[b],
                                 out_hbm.at[pl.ds(base + ch * CHUNK, CHUNK)],
                                 wsems[b])
        prev = ch - 1
        if prev >= 0 and prev + NBUF < NCH:
            bp = prev % NBUF
            wh[bp].wait()
            gh[bp] = pltpu.async_copy(
                table_hbm.at[idx_v.at[pl.ds((prev + NBUF) * CHUNK, CHUNK)]],
                rows_v.at[bp], gsems[bp])
    for ch in range(NCH - NBUF, NCH):
        if ch >= 0:
            wh[ch % NBUF].wait()


_sc_gather = functools.partial(
    pl.kernel,
    mesh=plsc.VectorSubcoreMesh(core_axis_name="c", subcore_axis_name="s"),
    out_type=jax.ShapeDtypeStruct((B_SC, EMB_DIM), jnp.float32),
    scratch_types=[
        pltpu.VMEM((B_PER_W,), jnp.int32),
        pltpu.VMEM((NBUF, CHUNK, EMB_DIM), jnp.float32),
        pltpu.SemaphoreType.DMA,
        pltpu.SemaphoreType.DMA,
        pltpu.SemaphoreType.DMA,
        pltpu.SemaphoreType.DMA,
        pltpu.SemaphoreType.DMA,
        pltpu.SemaphoreType.DMA,
    ],
)(_gather_body)


ROWS_PER_BLK = 512
NBLK = B_TC // ROWS_PER_BLK


def _tc_body(idx_ref, table_ref, out_ref):
    i = pl.program_id(0)

    def f(j, _):
        r = idx_ref[i * ROWS_PER_BLK + j]
        out_ref[j] = table_ref[r]
        return 0

    lax.fori_loop(0, ROWS_PER_BLK, f, 0, unroll=8)


_tc_gather = pl.pallas_call(
    _tc_body,
    grid_spec=pltpu.PrefetchScalarGridSpec(
        num_scalar_prefetch=1,
        grid=(NBLK,),
        in_specs=[
            pl.BlockSpec((NUM_EMB, 8, 128), lambda i, idx: (0, 0, 0)),
        ],
        out_specs=pl.BlockSpec((ROWS_PER_BLK, 8, 128),
                               lambda i, idx: (i, 0, 0)),
    ),
    out_shape=jax.ShapeDtypeStruct((B_TC, 8, 128), jnp.float32),
)


@jax.jit
def kernel(x, pos_encoding):
    sc_out = _sc_gather(x[:B_SC], pos_encoding)
    table = pos_encoding.reshape(NUM_EMB, 8, 128)
    tc_out = _tc_gather(x[B_SC:], table).reshape(B_TC, EMB_DIM)
    return jnp.concatenate([sc_out, tc_out], axis=0)
